# baseline (speedup 1.0000x reference)
import jax
import jax.numpy as jnp
from jax import lax
from jax.experimental import pallas as pl
from jax.experimental.pallas import tpu as pltpu
from jax.experimental.pallas import tpu_sc as plsc

_NC = 2
_NS = 16
_NW = _NC * _NS
_L = 16

_D = 64
_B = 4096
_S = 200
_BL = 128
_GROUPS = _BL // _L
_EPS = 1e-6


def _ln_block(rows_v, staging_v, gamma_v, beta_v, perms, dhi_c, dlo_c):
    gs = [gamma_v[pl.ds(k * _L, _L)] for k in range(4)]
    bs = [beta_v[pl.ds(k * _L, _L)] for k in range(4)]

    def group_body(g, _):
        rb = g * _L
        for i in range(_L):
            row = rb + i
            x = [rows_v[row, pl.ds(k * _L, _L)] for k in range(4)]
            s = (x[0] + x[1]) + (x[2] + x[3])
            q = (x[0] * x[0] + x[1] * x[1]) + (x[2] * x[2] + x[3] * x[3])
            for p in perms:
                s = s + s.at[p].get(mode="promise_in_bounds")
                q = q + q.at[p].get(mode="promise_in_bounds")
            mean = s * (1.0 / _D)
            var = q * (1.0 / _D) - mean * mean
            r = var + _EPS
            ib = plsc.bitcast(r, jnp.int32)
            ib = 0x5F3759DF - lax.shift_right_logical(ib, 1)
            y = plsc.bitcast(ib, jnp.float32)
            y = y * (1.5 - 0.5 * r * y * y)
            y = y * (1.5 - 0.5 * r * y * y)
            rvec = jnp.full((_L,), row, jnp.int32)
            for k in range(4):
                t = (x[k] - mean) * (y * gs[k]) + bs[k]
                plsc.store_scatter(staging_v, [dhi_c[k], dlo_c[k], rvec], t)
        return 0

    lax.fori_loop(0, _GROUPS, group_body, 0)


def _body(src_hbm, table_hbm, gamma_hbm, beta_hbm, out_hbm,
          idx_v, rows0, rows1, st0, st1, gamma_v, beta_v,
          gsem0, gsem1, wsem0, wsem1):
    wid = lax.axis_index("s") * _NC + lax.axis_index("c")
    pltpu.sync_copy(gamma_hbm, gamma_v)
    pltpu.sync_copy(beta_hbm, beta_v)
    pltpu.sync_copy(src_hbm.at[pl.ds(0, 25), wid], idx_v)

    rows = (rows0, rows1)
    stg = (st0, st1)
    gsems = (gsem0, gsem1)
    wsems = (wsem0, wsem1)
    lanes = lax.iota(jnp.int32, _L)
    perms = [lanes ^ sh for sh in (8, 4, 2, 1)]
    dhi_c = [(k * _L + lanes) >> 3 for k in range(4)]
    dlo_c = [(k * _L + lanes) & 7 for k in range(4)]

    def fire_gather(s, buf, sem):
        pltpu.async_copy(table_hbm.at[idx_v.at[s // 8, s % 8]], buf, sem)

    def drain_g(buf, sem):
        pltpu.make_async_copy(table_hbm.at[pl.ds(0, _BL)], buf, sem).wait()

    def drain_w(buf, sem):
        pltpu.make_async_copy(out_hbm.at[0, pl.ds(0, 8), 0], buf, sem).wait()

    fire_gather(0, rows0, gsem0)

    def super_body(sc, _):
        for b in range(2):
            s = sc * 2 + b
            nb = 1 - b

            @pl.when(s + 1 < _S)
            def _prefetch():
                @pl.when(s >= 1)
                def _recycle():
                    drain_w(stg[nb], wsems[nb])
                fire_gather(s + 1, rows[nb], gsems[nb])

            drain_g(rows[b], gsems[b])
            _ln_block(rows[b], stg[b], gamma_v, beta_v, perms, dhi_c, dlo_c)
            pltpu.async_copy(stg[b], out_hbm.at[s, pl.ds(0, 8), wid], wsems[b])
        return 0

    lax.fori_loop(0, _S // 2, super_body, 0)
    drain_w(st0, wsem0)
    drain_w(st1, wsem1)


@jax.jit
def _sc_lookup_ln(src4, table, gamma, beta):
    mesh = plsc.VectorSubcoreMesh(core_axis_name="c", subcore_axis_name="s")
    f = pl.kernel(
        _body,
        out_type=jax.ShapeDtypeStruct((_S, 8, _NW, 8, _BL), jnp.float32),
        mesh=mesh,
        scratch_types=[
            pltpu.VMEM((25, 8, _BL), jnp.int32),
            pltpu.VMEM((_BL, _D), jnp.float32),
            pltpu.VMEM((_BL, _D), jnp.float32),
            pltpu.VMEM((8, 8, _BL), jnp.float32),
            pltpu.VMEM((8, 8, _BL), jnp.float32),
            pltpu.VMEM((_D,), jnp.float32),
            pltpu.VMEM((_D,), jnp.float32),
            pltpu.SemaphoreType.DMA,
            pltpu.SemaphoreType.DMA,
            pltpu.SemaphoreType.DMA,
            pltpu.SemaphoreType.DMA,
        ],
        compiler_params=pltpu.CompilerParams(
            needs_layout_passes=False, use_tc_tiling_on_sc=False),
    )
    return f(src4, table, gamma, beta)


def kernel(src, word_embedding, ln_gamma, ln_beta):
    src4 = jnp.transpose(src.astype(jnp.int32).reshape(32, 128, 25, 8),
                         (2, 0, 3, 1))
    out5 = _sc_lookup_ln(src4, word_embedding, ln_gamma, ln_beta)
    return jnp.transpose(out5, (2, 4, 0, 1, 3)).reshape(_B, _S, _D)
